# baseline (speedup 1.0000x reference)
import functools

import jax
import jax.numpy as jnp
from jax import lax
from jax.experimental import pallas as pl
from jax.experimental.pallas import tpu as pltpu
from jax.experimental.pallas import tpu_sc as plsc

NC = 2
NS = 16
NW = NC * NS
CHUNK = 128


def _gather_body(bpw, u_tab, m_tab, uidx_hbm, midx_hbm, u_out, m_out,
                 uidx_v, midx_v, urows_v, mrows_v, sem):
    wid = lax.axis_index("s") * NC + lax.axis_index("c")
    nchunk = bpw // CHUNK
    pltpu.sync_copy(uidx_hbm.at[pl.ds(wid * nchunk, nchunk)], uidx_v)
    pltpu.sync_copy(midx_hbm.at[pl.ds(wid * nchunk, nchunk)], midx_v)
    copies = []
    for j in range(nchunk):
        c = pltpu.async_copy(u_tab.at[uidx_v.at[j]],
                             urows_v.at[pl.ds(j * CHUNK, CHUNK)], sem)
        copies.append(c)
        c = pltpu.async_copy(m_tab.at[midx_v.at[j]],
                             mrows_v.at[pl.ds(j * CHUNK, CHUNK)], sem)
        copies.append(c)
    for c in copies:
        c.wait()
    pltpu.sync_copy(urows_v, u_out.at[pl.ds(wid * bpw, bpw)])
    pltpu.sync_copy(mrows_v, m_out.at[pl.ds(wid * bpw, bpw)])


def _sc_gather(U, M, uidx, midx, batch):
    d = U.shape[1]
    bpw = batch // NW
    mesh = plsc.VectorSubcoreMesh(core_axis_name="c", subcore_axis_name="s")
    f = pl.kernel(
        functools.partial(_gather_body, bpw),
        out_type=(jax.ShapeDtypeStruct((batch, d), jnp.float32),
                  jax.ShapeDtypeStruct((batch, d), jnp.float32)),
        mesh=mesh,
        scratch_types=[
            pltpu.VMEM((bpw // CHUNK, CHUNK), jnp.int32),
            pltpu.VMEM((bpw // CHUNK, CHUNK), jnp.int32),
            pltpu.VMEM((bpw, d), jnp.float32),
            pltpu.VMEM((bpw, d), jnp.float32),
            pltpu.SemaphoreType.DMA,
        ],
        compiler_params=pltpu.CompilerParams(use_tc_tiling_on_sc=False),
    )
    return f(U, M, uidx, midx)


def _mlp_body(u_ref, m_ref, w1a_ref, w1b_ref, b1_ref, w2_ref, b2_ref,
              w3_ref, b3_ref, o_ref):
    h = (jnp.dot(u_ref[...], w1a_ref[...], preferred_element_type=jnp.float32)
         + jnp.dot(m_ref[...], w1b_ref[...], preferred_element_type=jnp.float32)
         + b1_ref[...])
    h = jnp.maximum(h, 0.0)
    h = jnp.dot(h, w2_ref[...], preferred_element_type=jnp.float32) + b2_ref[...]
    h = jnp.maximum(h, 0.0)
    z = jnp.sum(h * w3_ref[...], axis=1, keepdims=True) + b3_ref[...]
    o = 1.0 / (1.0 + jnp.exp(-z))
    o_ref[...] = o * 4.0 + 1.0


def _tc_mlp(u_emb, m_emb, w1a, w1b, b1, w2, b2, w3, b3, batch, blk):
    grid = (batch // blk,)
    full = lambda i: (0, 0)
    return pl.pallas_call(
        _mlp_body,
        grid=grid,
        in_specs=[
            pl.BlockSpec((blk, 64), lambda i: (i, 0)),
            pl.BlockSpec((blk, 64), lambda i: (i, 0)),
            pl.BlockSpec((64, 128), full),
            pl.BlockSpec((64, 128), full),
            pl.BlockSpec((1, 128), full),
            pl.BlockSpec((128, 128), full),
            pl.BlockSpec((1, 128), full),
            pl.BlockSpec((1, 128), full),
            pl.BlockSpec((1, 1), full),
        ],
        out_specs=pl.BlockSpec((blk, 1), lambda i: (i, 0)),
        out_shape=jax.ShapeDtypeStruct((batch, 1), jnp.float32),
    )(u_emb, m_emb, w1a, w1b, b1, w2, b2, w3, b3)


def kernel(users, movies, U, M, W1, b1, W2, b2, W3, b3):
    batch = users.shape[0]
    nf = U.shape[1]
    uidx = users.astype(jnp.int32).reshape(batch // CHUNK, CHUNK)
    midx = movies.astype(jnp.int32).reshape(batch // CHUNK, CHUNK)
    u_emb, m_emb = _sc_gather(U, M, uidx, midx, batch)
    w1a = W1[:, :nf].T
    w1b = W1[:, nf:].T
    out = _tc_mlp(u_emb, m_emb, w1a, w1b, b1.reshape(1, -1), W2.T,
                  b2.reshape(1, -1), W3, b3.reshape(1, 1), batch, 2048)
    return out
